# baseline (speedup 1.0000x reference)
import functools

import jax
import jax.numpy as jnp
from jax import lax
from jax.experimental import pallas as pl
from jax.experimental.pallas import tpu as pltpu
from jax.experimental.pallas import tpu_sc as plsc

EMB = 300
EMBP = 384
NCH = 3
HID = 600
HIDP = 640
N = 10000
NPAD = 10240
E = 160000
EPAD = 163840
NG = 256
OUT_DIM = 2048
NTILES = 16
RPT = NPAD // NTILES
NROWCH = RPT // 128
ECHUNK = EPAD // 2 // NTILES // 128
BN_EPS = 1e-5
BM = 1024

_MESH = plsc.VectorSubcoreMesh(core_axis_name="c", subcore_axis_name="s")


NBUF = 2


def _zero_fill(buf):
    def row(i, carry):
        for k in range(8):
            buf[i, pl.ds(k * 16, 16)] = jnp.zeros((16,), jnp.float32)
        return carry
    lax.fori_loop(0, 128, row, 0)


def _scatter_pass(table, sidx_v, didx_v, rows_v, agg_s, gsems, ssems, nchunk):
    def gxfer(j, b):
        return pltpu.make_async_copy(table.at[sidx_v.at[j]], rows_v.at[b],
                                     gsems[b])

    def sxfer(j, b):
        return pltpu.make_async_copy(rows_v.at[b], agg_s.at[didx_v.at[j]],
                                     ssems[b])

    for b in range(NBUF):
        gxfer(b, b).start()

    def group(g, carry):
        for b in range(NBUF):
            j = g * NBUF + b
            gxfer(j, b).wait()
            sxfer(j, b).start(add=True)
            sxfer(j, b).wait()

            @pl.when(j + NBUF < nchunk)
            def _():
                gxfer(j + NBUF, b).start()
        return carry

    lax.fori_loop(0, nchunk // NBUF, group, 0)


def _init_body(ctab0, ctab1, ctab2, cidx3, combo4, dst4, oh128,
               h0, h1, h2, hista, histb,
               cidx_v, rows_v, combo_v, didx_v,
               hist_s, *sems):
    gsems, ssems = sems[:NBUF], sems[NBUF:]
    c = lax.axis_index("c")
    s = lax.axis_index("s")
    base = s * RPT

    _zero_fill(rows_v.at[0])
    for r in range(NROWCH):
        pltpu.sync_copy(rows_v.at[0], hist_s.at[pl.ds(base + r * 128, 128)])

    pltpu.sync_copy(cidx3.at[s], cidx_v)

    def gather_rows(tab, out):
        def gx(r, b):
            return pltpu.make_async_copy(tab.at[cidx_v.at[r]], rows_v.at[b],
                                         gsems[b])

        def wx(r, b):
            return pltpu.make_async_copy(
                rows_v.at[b], out.at[pl.ds(base + r * 128, 128)], ssems[b])

        gx(0, 0).start()
        for r in range(NROWCH):
            b = r % NBUF
            gx(r, b).wait()
            if r + 1 < NROWCH:
                gx(r + 1, (r + 1) % NBUF).start()
            wx(r, b).start()
            wx(r, b).wait()

    @pl.when(c == 0)
    def _():
        gather_rows(ctab0, h0)
        gather_rows(ctab1, h1)

    @pl.when(c == 1)
    def _():
        gather_rows(ctab2, h2)

    pltpu.sync_copy(combo4.at[c, s], combo_v)
    pltpu.sync_copy(dst4.at[c, s], didx_v)
    plsc.subcore_barrier()

    _scatter_pass(oh128, combo_v, didx_v, rows_v, hist_s, gsems, ssems,
                  ECHUNK)
    plsc.subcore_barrier()

    @pl.when(c == 0)
    def _():
        pltpu.sync_copy(hist_s.at[pl.ds(base, RPT)], hista.at[pl.ds(base, RPT)])

    @pl.when(c == 1)
    def _():
        pltpu.sync_copy(hist_s.at[pl.ds(base, RPT)], histb.at[pl.ds(base, RPT)])


_init_kernel = pl.kernel(
    _init_body,
    out_type=(
        jax.ShapeDtypeStruct((NPAD, 128), jnp.float32),
        jax.ShapeDtypeStruct((NPAD, 128), jnp.float32),
        jax.ShapeDtypeStruct((NPAD, 128), jnp.float32),
        jax.ShapeDtypeStruct((NPAD, 128), jnp.float32),
        jax.ShapeDtypeStruct((NPAD, 128), jnp.float32),
    ),
    mesh=_MESH,
    scratch_types=[
        pltpu.VMEM((NROWCH, 128), jnp.int32),
        pltpu.VMEM((NBUF, 128, 128), jnp.float32),
        pltpu.VMEM((ECHUNK, 128), jnp.int32),
        pltpu.VMEM((ECHUNK, 128), jnp.int32),
        pltpu.VMEM_SHARED((NPAD, 128), jnp.float32),
    ] + [pltpu.SemaphoreType.DMA] * (2 * NBUF),
)


def _spmm_body(h0, h1, h2, src4, dst4,
               oa0, ob0, oa1, ob1, oa2, ob2,
               sidx_v, didx_v, rows_v, agg_s, *sems):
    gsems, ssems = sems[:NBUF], sems[NBUF:]
    c = lax.axis_index("c")
    s = lax.axis_index("s")
    base = s * RPT

    pltpu.sync_copy(src4.at[c, s], sidx_v)
    pltpu.sync_copy(dst4.at[c, s], didx_v)

    def chunk_pass(h_k, outa_k, outb_k):
        @pl.when(c == 0)
        def _():
            pltpu.sync_copy(h_k.at[pl.ds(base, RPT)],
                            agg_s.at[pl.ds(base, RPT)])

        @pl.when(c == 1)
        def _():
            _zero_fill(rows_v.at[0])
            for r in range(NROWCH):
                pltpu.sync_copy(rows_v.at[0],
                                agg_s.at[pl.ds(base + r * 128, 128)])
        plsc.subcore_barrier()

        _scatter_pass(h_k, sidx_v, didx_v, rows_v, agg_s, gsems, ssems,
                      ECHUNK)
        plsc.subcore_barrier()

        @pl.when(c == 0)
        def _():
            pltpu.sync_copy(agg_s.at[pl.ds(base, RPT)],
                            outa_k.at[pl.ds(base, RPT)])

        @pl.when(c == 1)
        def _():
            pltpu.sync_copy(agg_s.at[pl.ds(base, RPT)],
                            outb_k.at[pl.ds(base, RPT)])
        plsc.subcore_barrier()

    chunk_pass(h0, oa0, ob0)
    chunk_pass(h1, oa1, ob1)
    chunk_pass(h2, oa2, ob2)


_spmm_kernel = pl.kernel(
    _spmm_body,
    out_type=tuple(jax.ShapeDtypeStruct((NPAD, 128), jnp.float32)
                   for _ in range(6)),
    mesh=_MESH,
    scratch_types=[
        pltpu.VMEM((ECHUNK, 128), jnp.int32),
        pltpu.VMEM((ECHUNK, 128), jnp.int32),
        pltpu.VMEM((NBUF, 128, 128), jnp.float32),
        pltpu.VMEM_SHARED((NPAD, 128), jnp.float32),
    ] + [pltpu.SemaphoreType.DMA] * (2 * NBUF),
)


def _mlp_body(sa0, sb0, sa1, sb1, sa2, sb2, ha_ref, hb_ref,
              w10, w11, w12, ew1, b1, w20, w21, w22, b20, b21, b22,
              o0, o1, o2, *, relu_out):
    f32 = jnp.float32
    hi = lax.Precision.HIGHEST
    x1 = jnp.dot(sa0[...] + sb0[...], w10[...], preferred_element_type=f32,
                 precision=hi)
    x1 = x1 + jnp.dot(sa1[...] + sb1[...], w11[...],
                      preferred_element_type=f32, precision=hi)
    x1 = x1 + jnp.dot(sa2[...] + sb2[...], w12[...],
                      preferred_element_type=f32, precision=hi)
    x1 = x1 + jnp.dot(ha_ref[...] + hb_ref[...], ew1[...],
                      preferred_element_type=f32, precision=hi)
    z1 = jnp.maximum(x1 + b1[...], 0.0)
    for w2, b2, o in ((w20, b20, o0), (w21, b21, o1), (w22, b22, o2)):
        z = jnp.dot(z1, w2[...], preferred_element_type=f32,
                    precision=hi) + b2[...]
        if relu_out:
            z = jnp.maximum(z, 0.0)
        o[...] = z


def _mlp_call(relu_out):
    nblk = NPAD // BM
    blk = pl.BlockSpec((BM, 128), lambda i: (i, 0))
    full = lambda shape: pl.BlockSpec(shape, lambda i: (0, 0))
    return pl.pallas_call(
        functools.partial(_mlp_body, relu_out=relu_out),
        grid=(nblk,),
        in_specs=[blk] * 8 + [
            full((128, HIDP)), full((128, HIDP)), full((128, HIDP)),
            full((128, HIDP)), full((1, HIDP)),
            full((HIDP, 128)), full((HIDP, 128)), full((HIDP, 128)),
            full((1, 128)), full((1, 128)), full((1, 128)),
        ],
        out_specs=[blk, blk, blk],
        out_shape=[jax.ShapeDtypeStruct((NPAD, 128), jnp.float32)
                   for _ in range(3)],
    )


def _head_body(h0, h1, h2, gid2, hw_ref, hbias, out, acc):
    i = pl.program_id(0)
    g = gid2[...]
    oh = (g == lax.broadcasted_iota(jnp.int32, (1, NG), 1)).astype(jnp.float32)
    x = jnp.concatenate(
        [h0[...], h1[...], h2[...], jnp.ones((BM, 128), jnp.float32)], axis=1)
    p = lax.dot_general(oh, x, (((0,), (0,)), ((), ())),
                        preferred_element_type=jnp.float32,
                        precision=lax.Precision.HIGHEST)

    @pl.when(i == 0)
    def _():
        acc[...] = p

    @pl.when(i > 0)
    def _():
        acc[...] = acc[...] + p

    @pl.when(i == pl.num_programs(0) - 1)
    def _():
        a = acc[...]
        cnt = a[:, EMBP:EMBP + 1]
        pooled = a[:, :EMBP] / jnp.maximum(cnt, 1.0)
        out[...] = jnp.dot(pooled, hw_ref[...],
                           preferred_element_type=jnp.float32,
                           precision=lax.Precision.HIGHEST) + hbias[...]


_head_kernel = pl.pallas_call(
    _head_body,
    grid=(NPAD // BM,),
    in_specs=[
        pl.BlockSpec((BM, 128), lambda i: (i, 0)),
        pl.BlockSpec((BM, 128), lambda i: (i, 0)),
        pl.BlockSpec((BM, 128), lambda i: (i, 0)),
        pl.BlockSpec((BM, 1), lambda i: (i, 0)),
        pl.BlockSpec((EMBP, OUT_DIM), lambda i: (0, 0)),
        pl.BlockSpec((1, OUT_DIM), lambda i: (0, 0)),
    ],
    out_specs=pl.BlockSpec((NG, OUT_DIM), lambda i: (0, 0)),
    out_shape=jax.ShapeDtypeStruct((NG, OUT_DIM), jnp.float32),
    scratch_shapes=[pltpu.VMEM((NG, EMBP + 128), jnp.float32)],
)


def kernel(params, atom_type, chirality, edge_index, bond_type,
           bond_direction, graph_ids):
    f32 = jnp.float32

    atom_emb = params['atom_emb'].astype(f32)
    chi_emb = params['chi_emb'].astype(f32)
    ctab = (atom_emb[:, None, :] + chi_emb[None, :, :]).reshape(360, EMB)
    ctab = jnp.pad(ctab, ((0, 8), (0, EMBP - EMB)))
    ctabs = [ctab[:, k * 128:(k + 1) * 128] for k in range(NCH)]
    oh128 = jnp.pad(jnp.eye(32, dtype=f32), ((0, 0), (0, 96)))

    w1_l, ew1_l, b1_l, w2_l, b2_l = [], [], [], [], []
    for p in params['layers']:
        w1 = jnp.pad(p['W1'].astype(f32),
                     ((0, EMBP - EMB), (0, HIDP - HID)))
        w1_l.append([w1[k * 128:(k + 1) * 128] for k in range(NCH)])
        ec = (p['e_bond'][:, None, :] + p['e_dir'][None, :, :]).reshape(18, EMB)
        ew1 = ec.astype(f32) @ p['W1'].astype(f32)
        ew1_l.append(jnp.pad(ew1, ((0, 128 - 18), (0, HIDP - HID))))
        b1_l.append(jnp.pad(p['b1'].astype(f32), (0, HIDP - HID))[None, :])
        scale = p['bn_g'].astype(f32) / jnp.sqrt(1.0 + BN_EPS)
        w2f = p['W2'].astype(f32) * scale[None, :]
        b2f = p['b2'].astype(f32) * scale + p['bn_b'].astype(f32)
        w2f = jnp.pad(w2f, ((0, HIDP - HID), (0, EMBP - EMB)))
        b2f = jnp.pad(b2f, (0, EMBP - EMB))
        w2_l.append([w2f[:, k * 128:(k + 1) * 128] for k in range(NCH)])
        b2_l.append([b2f[None, k * 128:(k + 1) * 128] for k in range(NCH)])

    hw_p = jnp.pad(params['head_W'].astype(f32), ((0, EMBP - EMB), (0, 0)))
    hb_p = params['head_b'].astype(f32)[None, :]

    i32 = jnp.int32
    cidx = atom_type.astype(i32) * 3 + chirality.astype(i32)
    cidx3 = jnp.pad(cidx, (0, NPAD - N)).reshape(NTILES, NROWCH, 128)
    esh = (2, NTILES, ECHUNK, 128)
    src4 = jnp.pad(edge_index[0].astype(i32), (0, EPAD - E),
                   constant_values=NPAD - 1).reshape(esh)
    dst4 = jnp.pad(edge_index[1].astype(i32), (0, EPAD - E),
                   constant_values=NPAD - 1).reshape(esh)
    combo = bond_type.astype(i32) * 3 + bond_direction.astype(i32)
    combo4 = jnp.pad(combo, (0, EPAD - E)).reshape(esh)
    gid2 = jnp.pad(graph_ids.astype(i32), (0, NPAD - N),
                   constant_values=NG).reshape(NPAD, 1)

    h0, h1, h2, hista, histb = _init_kernel(*ctabs, cidx3, combo4, dst4,
                                            oh128)

    for l in range(5):
        parts = _spmm_kernel(h0, h1, h2, src4, dst4)
        h0, h1, h2 = _mlp_call(relu_out=(l != 4))(
            *parts, hista, histb, *w1_l[l], ew1_l[l], b1_l[l],
            *w2_l[l], *b2_l[l])

    return _head_kernel(h0, h1, h2, gid2, hw_p, hb_p)


# speedup vs baseline: 1.0974x; 1.0974x over previous
import functools

import jax
import jax.numpy as jnp
from jax import lax
from jax.experimental import pallas as pl
from jax.experimental.pallas import tpu as pltpu
from jax.experimental.pallas import tpu_sc as plsc

EMB = 300
EMBP = 384
NCH = 3
HID = 600
HIDP = 640
N = 10000
NPAD = 10240
E = 160000
EPAD = 163840
NG = 256
OUT_DIM = 2048
NTILES = 16
RPT = NPAD // NTILES
NROWCH = RPT // 128
ECHUNK = EPAD // 2 // NTILES // 128
BN_EPS = 1e-5
BM = 1024

_MESH = plsc.VectorSubcoreMesh(core_axis_name="c", subcore_axis_name="s")


NBUF = 2


def _zero_fill(buf):
    def row(i, carry):
        for k in range(8):
            buf[i, pl.ds(k * 16, 16)] = jnp.zeros((16,), jnp.float32)
        return carry
    lax.fori_loop(0, 128, row, 0)


def _scatter_pass(table, sidx_v, didx_v, rows_v, agg_s, gsems, ssems, nchunk):
    def gxfer(j, b):
        return pltpu.make_async_copy(table.at[sidx_v.at[j]], rows_v.at[b],
                                     gsems[b])

    def sxfer(j, b):
        return pltpu.make_async_copy(rows_v.at[b], agg_s.at[didx_v.at[j]],
                                     ssems[b])

    for b in range(NBUF):
        gxfer(b, b).start()

    def group(g, carry):
        for b in range(NBUF):
            j = g * NBUF + b
            gxfer(j, b).wait()
            sxfer(j, b).start(add=True)
            sxfer(j, b).wait()

            @pl.when(j + NBUF < nchunk)
            def _():
                gxfer(j + NBUF, b).start()
        return carry

    lax.fori_loop(0, nchunk // NBUF, group, 0)


def _init_body(ctab0, ctab1, ctab2, cidx3, combo4, dst4, oh128,
               h0, h1, h2, hista, histb,
               cidx_v, rows_v, combo_v, didx_v,
               hist_s, oh_s, *sems):
    gsems, ssems = sems[:NBUF], sems[NBUF:]
    c = lax.axis_index("c")
    s = lax.axis_index("s")
    base = s * RPT

    @pl.when(s == 0)
    def _():
        pltpu.sync_copy(oh128, oh_s)

    _zero_fill(rows_v.at[0])
    for r in range(NROWCH):
        pltpu.sync_copy(rows_v.at[0], hist_s.at[pl.ds(base + r * 128, 128)])

    pltpu.sync_copy(cidx3.at[s], cidx_v)

    def gather_rows(tab, out):
        def gx(r, b):
            return pltpu.make_async_copy(tab.at[cidx_v.at[r]], rows_v.at[b],
                                         gsems[b])

        def wx(r, b):
            return pltpu.make_async_copy(
                rows_v.at[b], out.at[pl.ds(base + r * 128, 128)], ssems[b])

        gx(0, 0).start()
        for r in range(NROWCH):
            b = r % NBUF
            gx(r, b).wait()
            if r + 1 < NROWCH:
                gx(r + 1, (r + 1) % NBUF).start()
            wx(r, b).start()
            wx(r, b).wait()

    @pl.when(c == 0)
    def _():
        gather_rows(ctab0, h0)
        gather_rows(ctab1, h1)

    @pl.when(c == 1)
    def _():
        gather_rows(ctab2, h2)

    pltpu.sync_copy(combo4.at[c, s], combo_v)
    pltpu.sync_copy(dst4.at[c, s], didx_v)
    plsc.subcore_barrier()

    _scatter_pass(oh_s, combo_v, didx_v, rows_v, hist_s, gsems, ssems,
                  ECHUNK)
    plsc.subcore_barrier()

    @pl.when(c == 0)
    def _():
        pltpu.sync_copy(hist_s.at[pl.ds(base, RPT)], hista.at[pl.ds(base, RPT)])

    @pl.when(c == 1)
    def _():
        pltpu.sync_copy(hist_s.at[pl.ds(base, RPT)], histb.at[pl.ds(base, RPT)])


_init_kernel = pl.kernel(
    _init_body,
    out_type=(
        jax.ShapeDtypeStruct((NPAD, 128), jnp.float32),
        jax.ShapeDtypeStruct((NPAD, 128), jnp.float32),
        jax.ShapeDtypeStruct((NPAD, 128), jnp.float32),
        jax.ShapeDtypeStruct((NPAD, 128), jnp.float32),
        jax.ShapeDtypeStruct((NPAD, 128), jnp.float32),
    ),
    mesh=_MESH,
    scratch_types=[
        pltpu.VMEM((NROWCH, 128), jnp.int32),
        pltpu.VMEM((NBUF, 128, 128), jnp.float32),
        pltpu.VMEM((ECHUNK, 128), jnp.int32),
        pltpu.VMEM((ECHUNK, 128), jnp.int32),
        pltpu.VMEM_SHARED((NPAD, 128), jnp.float32),
        pltpu.VMEM_SHARED((32, 128), jnp.float32),
    ] + [pltpu.SemaphoreType.DMA] * (2 * NBUF),
)


def _spmm_body(h0, h1, h2, src4, dst4,
               oa0, ob0, oa1, ob1, oa2, ob2,
               sidx_v, didx_v, rows_v, agg_s, *sems):
    gsems, ssems = sems[:NBUF], sems[NBUF:]
    c = lax.axis_index("c")
    s = lax.axis_index("s")
    base = s * RPT

    pltpu.sync_copy(src4.at[c, s], sidx_v)
    pltpu.sync_copy(dst4.at[c, s], didx_v)

    def chunk_pass(h_k, outa_k, outb_k):
        @pl.when(c == 0)
        def _():
            pltpu.sync_copy(h_k.at[pl.ds(base, RPT)],
                            agg_s.at[pl.ds(base, RPT)])

        @pl.when(c == 1)
        def _():
            _zero_fill(rows_v.at[0])
            for r in range(NROWCH):
                pltpu.sync_copy(rows_v.at[0],
                                agg_s.at[pl.ds(base + r * 128, 128)])
        plsc.subcore_barrier()

        _scatter_pass(h_k, sidx_v, didx_v, rows_v, agg_s, gsems, ssems,
                      ECHUNK)
        plsc.subcore_barrier()

        @pl.when(c == 0)
        def _():
            pltpu.sync_copy(agg_s.at[pl.ds(base, RPT)],
                            outa_k.at[pl.ds(base, RPT)])

        @pl.when(c == 1)
        def _():
            pltpu.sync_copy(agg_s.at[pl.ds(base, RPT)],
                            outb_k.at[pl.ds(base, RPT)])
        plsc.subcore_barrier()

    chunk_pass(h0, oa0, ob0)
    chunk_pass(h1, oa1, ob1)
    chunk_pass(h2, oa2, ob2)


_spmm_kernel = pl.kernel(
    _spmm_body,
    out_type=tuple(jax.ShapeDtypeStruct((NPAD, 128), jnp.float32)
                   for _ in range(6)),
    mesh=_MESH,
    scratch_types=[
        pltpu.VMEM((ECHUNK, 128), jnp.int32),
        pltpu.VMEM((ECHUNK, 128), jnp.int32),
        pltpu.VMEM((NBUF, 128, 128), jnp.float32),
        pltpu.VMEM_SHARED((NPAD, 128), jnp.float32),
    ] + [pltpu.SemaphoreType.DMA] * (2 * NBUF),
)


def _mlp_body(sa0, sb0, sa1, sb1, sa2, sb2, ha_ref, hb_ref,
              w10, w11, w12, ew1, b1, w20, w21, w22, b20, b21, b22,
              o0, o1, o2, *, relu_out):
    f32 = jnp.float32
    hi = lax.Precision.HIGHEST
    x1 = jnp.dot(sa0[...] + sb0[...], w10[...], preferred_element_type=f32,
                 precision=hi)
    x1 = x1 + jnp.dot(sa1[...] + sb1[...], w11[...],
                      preferred_element_type=f32, precision=hi)
    x1 = x1 + jnp.dot(sa2[...] + sb2[...], w12[...],
                      preferred_element_type=f32, precision=hi)
    x1 = x1 + jnp.dot(ha_ref[...] + hb_ref[...], ew1[...],
                      preferred_element_type=f32, precision=hi)
    z1 = jnp.maximum(x1 + b1[...], 0.0)
    for w2, b2, o in ((w20, b20, o0), (w21, b21, o1), (w22, b22, o2)):
        z = jnp.dot(z1, w2[...], preferred_element_type=f32,
                    precision=hi) + b2[...]
        if relu_out:
            z = jnp.maximum(z, 0.0)
        o[...] = z


def _mlp_call(relu_out):
    nblk = NPAD // BM
    blk = pl.BlockSpec((BM, 128), lambda i: (i, 0))
    full = lambda shape: pl.BlockSpec(shape, lambda i: (0, 0))
    return pl.pallas_call(
        functools.partial(_mlp_body, relu_out=relu_out),
        grid=(nblk,),
        in_specs=[blk] * 8 + [
            full((128, HIDP)), full((128, HIDP)), full((128, HIDP)),
            full((128, HIDP)), full((1, HIDP)),
            full((HIDP, 128)), full((HIDP, 128)), full((HIDP, 128)),
            full((1, 128)), full((1, 128)), full((1, 128)),
        ],
        out_specs=[blk, blk, blk],
        out_shape=[jax.ShapeDtypeStruct((NPAD, 128), jnp.float32)
                   for _ in range(3)],
    )


def _head_body(h0, h1, h2, gid2, hw_ref, hbias, out, acc):
    i = pl.program_id(0)
    g = gid2[...]
    oh = (g == lax.broadcasted_iota(jnp.int32, (1, NG), 1)).astype(jnp.float32)
    x = jnp.concatenate(
        [h0[...], h1[...], h2[...], jnp.ones((BM, 128), jnp.float32)], axis=1)
    p = lax.dot_general(oh, x, (((0,), (0,)), ((), ())),
                        preferred_element_type=jnp.float32,
                        precision=lax.Precision.HIGHEST)

    @pl.when(i == 0)
    def _():
        acc[...] = p

    @pl.when(i > 0)
    def _():
        acc[...] = acc[...] + p

    @pl.when(i == pl.num_programs(0) - 1)
    def _():
        a = acc[...]
        cnt = a[:, EMBP:EMBP + 1]
        pooled = a[:, :EMBP] / jnp.maximum(cnt, 1.0)
        out[...] = jnp.dot(pooled, hw_ref[...],
                           preferred_element_type=jnp.float32,
                           precision=lax.Precision.HIGHEST) + hbias[...]


_head_kernel = pl.pallas_call(
    _head_body,
    grid=(NPAD // BM,),
    in_specs=[
        pl.BlockSpec((BM, 128), lambda i: (i, 0)),
        pl.BlockSpec((BM, 128), lambda i: (i, 0)),
        pl.BlockSpec((BM, 128), lambda i: (i, 0)),
        pl.BlockSpec((BM, 1), lambda i: (i, 0)),
        pl.BlockSpec((EMBP, OUT_DIM), lambda i: (0, 0)),
        pl.BlockSpec((1, OUT_DIM), lambda i: (0, 0)),
    ],
    out_specs=pl.BlockSpec((NG, OUT_DIM), lambda i: (0, 0)),
    out_shape=jax.ShapeDtypeStruct((NG, OUT_DIM), jnp.float32),
    scratch_shapes=[pltpu.VMEM((NG, EMBP + 128), jnp.float32)],
)


def kernel(params, atom_type, chirality, edge_index, bond_type,
           bond_direction, graph_ids):
    f32 = jnp.float32

    atom_emb = params['atom_emb'].astype(f32)
    chi_emb = params['chi_emb'].astype(f32)
    ctab = (atom_emb[:, None, :] + chi_emb[None, :, :]).reshape(360, EMB)
    ctab = jnp.pad(ctab, ((0, 8), (0, EMBP - EMB)))
    ctabs = [ctab[:, k * 128:(k + 1) * 128] for k in range(NCH)]
    oh128 = jnp.pad(jnp.eye(32, dtype=f32), ((0, 0), (0, 96)))

    w1_l, ew1_l, b1_l, w2_l, b2_l = [], [], [], [], []
    for p in params['layers']:
        w1 = jnp.pad(p['W1'].astype(f32),
                     ((0, EMBP - EMB), (0, HIDP - HID)))
        w1_l.append([w1[k * 128:(k + 1) * 128] for k in range(NCH)])
        ec = (p['e_bond'][:, None, :] + p['e_dir'][None, :, :]).reshape(18, EMB)
        ew1 = ec.astype(f32) @ p['W1'].astype(f32)
        ew1_l.append(jnp.pad(ew1, ((0, 128 - 18), (0, HIDP - HID))))
        b1_l.append(jnp.pad(p['b1'].astype(f32), (0, HIDP - HID))[None, :])
        scale = p['bn_g'].astype(f32) / jnp.sqrt(1.0 + BN_EPS)
        w2f = p['W2'].astype(f32) * scale[None, :]
        b2f = p['b2'].astype(f32) * scale + p['bn_b'].astype(f32)
        w2f = jnp.pad(w2f, ((0, HIDP - HID), (0, EMBP - EMB)))
        b2f = jnp.pad(b2f, (0, EMBP - EMB))
        w2_l.append([w2f[:, k * 128:(k + 1) * 128] for k in range(NCH)])
        b2_l.append([b2f[None, k * 128:(k + 1) * 128] for k in range(NCH)])

    hw_p = jnp.pad(params['head_W'].astype(f32), ((0, EMBP - EMB), (0, 0)))
    hb_p = params['head_b'].astype(f32)[None, :]

    i32 = jnp.int32
    cidx = atom_type.astype(i32) * 3 + chirality.astype(i32)
    cidx3 = jnp.pad(cidx, (0, NPAD - N)).reshape(NTILES, NROWCH, 128)
    esh = (2, NTILES, ECHUNK, 128)
    src4 = jnp.pad(edge_index[0].astype(i32), (0, EPAD - E),
                   constant_values=NPAD - 1).reshape(esh)
    dst4 = jnp.pad(edge_index[1].astype(i32), (0, EPAD - E),
                   constant_values=NPAD - 1).reshape(esh)
    combo = bond_type.astype(i32) * 3 + bond_direction.astype(i32)
    combo4 = jnp.pad(combo, (0, EPAD - E)).reshape(esh)
    gid2 = jnp.pad(graph_ids.astype(i32), (0, NPAD - N),
                   constant_values=NG).reshape(NPAD, 1)

    h0, h1, h2, hista, histb = _init_kernel(*ctabs, cidx3, combo4, dst4,
                                            oh128)

    for l in range(5):
        parts = _spmm_kernel(h0, h1, h2, src4, dst4)
        h0, h1, h2 = _mlp_call(relu_out=(l != 4))(
            *parts, hista, histb, *w1_l[l], ew1_l[l], b1_l[l],
            *w2_l[l], *b2_l[l])

    return _head_kernel(h0, h1, h2, gid2, hw_p, hb_p)


# speedup vs baseline: 1.0980x; 1.0006x over previous
import functools

import jax
import jax.numpy as jnp
from jax import lax
from jax.experimental import pallas as pl
from jax.experimental.pallas import tpu as pltpu
from jax.experimental.pallas import tpu_sc as plsc

EMB = 300
EMBP = 384
NCH = 3
HID = 600
HIDP = 640
N = 10000
NPAD = 10240
E = 160000
EPAD = 163840
NG = 256
OUT_DIM = 2048
NTILES = 16
RPT = NPAD // NTILES
CHR = 128
NROWCH = RPT // CHR
ECHUNK = EPAD // 2 // NTILES // CHR
BN_EPS = 1e-5
BM = 1024

_MESH = plsc.VectorSubcoreMesh(core_axis_name="c", subcore_axis_name="s")


NBUF = 2


def _zero_fill(buf):
    def row(i, carry):
        for k in range(8):
            buf[i, pl.ds(k * 16, 16)] = jnp.zeros((16,), jnp.float32)
        return carry
    lax.fori_loop(0, CHR, row, 0)


def _scatter_pass(table, sidx_v, didx_v, rows_v, agg_s, gsems, ssems, nchunk):
    def gxfer(j, b):
        return pltpu.make_async_copy(table.at[sidx_v.at[j]], rows_v.at[b],
                                     gsems[b])

    def sxfer(j, b):
        return pltpu.make_async_copy(rows_v.at[b], agg_s.at[didx_v.at[j]],
                                     ssems[b])

    for b in range(NBUF):
        gxfer(b, b).start()

    def group(g, carry):
        for b in range(NBUF):
            j = g * NBUF + b
            gxfer(j, b).wait()
            sxfer(j, b).start(add=True)
            sxfer(j, b).wait()

            @pl.when(j + NBUF < nchunk)
            def _():
                gxfer(j + NBUF, b).start()
        return carry

    lax.fori_loop(0, nchunk // NBUF, group, 0)


def _init_body(ctab0, ctab1, ctab2, cidx3, combo4, dst4, oh128,
               h0, h1, h2, hista, histb,
               cidx_v, rows_v, combo_v, didx_v,
               hist_s, oh_s, *sems):
    gsems, ssems = sems[:NBUF], sems[NBUF:]
    c = lax.axis_index("c")
    s = lax.axis_index("s")
    base = s * RPT

    @pl.when(s == 0)
    def _():
        pltpu.sync_copy(oh128, oh_s)

    _zero_fill(rows_v.at[0])
    for r in range(NROWCH):
        pltpu.sync_copy(rows_v.at[0], hist_s.at[pl.ds(base + r * 128, 128)])

    pltpu.sync_copy(cidx3.at[s], cidx_v)

    def gather_rows(tab, out):
        def gx(r, b):
            return pltpu.make_async_copy(tab.at[cidx_v.at[r]], rows_v.at[b],
                                         gsems[b])

        def wx(r, b):
            return pltpu.make_async_copy(
                rows_v.at[b], out.at[pl.ds(base + r * 128, 128)], ssems[b])

        gx(0, 0).start()
        for r in range(NROWCH):
            b = r % NBUF
            gx(r, b).wait()
            if r + 1 < NROWCH:
                gx(r + 1, (r + 1) % NBUF).start()
            wx(r, b).start()
            wx(r, b).wait()

    @pl.when(c == 0)
    def _():
        gather_rows(ctab0, h0)
        gather_rows(ctab1, h1)

    @pl.when(c == 1)
    def _():
        gather_rows(ctab2, h2)

    pltpu.sync_copy(combo4.at[c, s], combo_v)
    pltpu.sync_copy(dst4.at[c, s], didx_v)
    plsc.subcore_barrier()

    _scatter_pass(oh_s, combo_v, didx_v, rows_v, hist_s, gsems, ssems,
                  ECHUNK)
    plsc.subcore_barrier()

    @pl.when(c == 0)
    def _():
        pltpu.sync_copy(hist_s.at[pl.ds(base, RPT)], hista.at[pl.ds(base, RPT)])

    @pl.when(c == 1)
    def _():
        pltpu.sync_copy(hist_s.at[pl.ds(base, RPT)], histb.at[pl.ds(base, RPT)])


_init_kernel = pl.kernel(
    _init_body,
    out_type=(
        jax.ShapeDtypeStruct((NPAD, 128), jnp.float32),
        jax.ShapeDtypeStruct((NPAD, 128), jnp.float32),
        jax.ShapeDtypeStruct((NPAD, 128), jnp.float32),
        jax.ShapeDtypeStruct((NPAD, 128), jnp.float32),
        jax.ShapeDtypeStruct((NPAD, 128), jnp.float32),
    ),
    mesh=_MESH,
    scratch_types=[
        pltpu.VMEM((NROWCH, 128), jnp.int32),
        pltpu.VMEM((NBUF, 128, 128), jnp.float32),
        pltpu.VMEM((ECHUNK, 128), jnp.int32),
        pltpu.VMEM((ECHUNK, 128), jnp.int32),
        pltpu.VMEM_SHARED((NPAD, 128), jnp.float32),
        pltpu.VMEM_SHARED((32, 128), jnp.float32),
    ] + [pltpu.SemaphoreType.DMA] * (2 * NBUF),
)


def _spmm_body(h0, h1, h2, src4, dst4,
               oa0, ob0, oa1, ob1, oa2, ob2,
               sidx_v, didx_v, rows_v, agg_s, *sems):
    gsems, ssems = sems[:NBUF], sems[NBUF:]
    c = lax.axis_index("c")
    s = lax.axis_index("s")
    base = s * RPT

    pltpu.sync_copy(src4.at[c, s], sidx_v)
    pltpu.sync_copy(dst4.at[c, s], didx_v)

    def chunk_pass(h_k, outa_k, outb_k):
        @pl.when(c == 0)
        def _():
            pltpu.sync_copy(h_k.at[pl.ds(base, RPT)],
                            agg_s.at[pl.ds(base, RPT)])

        @pl.when(c == 1)
        def _():
            _zero_fill(rows_v.at[0])
            for r in range(NROWCH):
                pltpu.sync_copy(rows_v.at[0],
                                agg_s.at[pl.ds(base + r * 128, 128)])
        plsc.subcore_barrier()

        _scatter_pass(h_k, sidx_v, didx_v, rows_v, agg_s, gsems, ssems,
                      ECHUNK)
        plsc.subcore_barrier()

        @pl.when(c == 0)
        def _():
            pltpu.sync_copy(agg_s.at[pl.ds(base, RPT)],
                            outa_k.at[pl.ds(base, RPT)])

        @pl.when(c == 1)
        def _():
            pltpu.sync_copy(agg_s.at[pl.ds(base, RPT)],
                            outb_k.at[pl.ds(base, RPT)])
        plsc.subcore_barrier()

    chunk_pass(h0, oa0, ob0)
    chunk_pass(h1, oa1, ob1)
    chunk_pass(h2, oa2, ob2)


_spmm_kernel = pl.kernel(
    _spmm_body,
    out_type=tuple(jax.ShapeDtypeStruct((NPAD, 128), jnp.float32)
                   for _ in range(6)),
    mesh=_MESH,
    scratch_types=[
        pltpu.VMEM((ECHUNK, 128), jnp.int32),
        pltpu.VMEM((ECHUNK, 128), jnp.int32),
        pltpu.VMEM((NBUF, 128, 128), jnp.float32),
        pltpu.VMEM_SHARED((NPAD, 128), jnp.float32),
    ] + [pltpu.SemaphoreType.DMA] * (2 * NBUF),
)


def _mlp_body(sa0, sb0, sa1, sb1, sa2, sb2, ha_ref, hb_ref,
              w10, w11, w12, ew1, b1, w20, w21, w22, b20, b21, b22,
              o0, o1, o2, *, relu_out):
    f32 = jnp.float32
    hi = lax.Precision.HIGHEST
    x1 = jnp.dot(sa0[...] + sb0[...], w10[...], preferred_element_type=f32,
                 precision=hi)
    x1 = x1 + jnp.dot(sa1[...] + sb1[...], w11[...],
                      preferred_element_type=f32, precision=hi)
    x1 = x1 + jnp.dot(sa2[...] + sb2[...], w12[...],
                      preferred_element_type=f32, precision=hi)
    x1 = x1 + jnp.dot(ha_ref[...] + hb_ref[...], ew1[...],
                      preferred_element_type=f32, precision=hi)
    z1 = jnp.maximum(x1 + b1[...], 0.0)
    for w2, b2, o in ((w20, b20, o0), (w21, b21, o1), (w22, b22, o2)):
        z = jnp.dot(z1, w2[...], preferred_element_type=f32,
                    precision=hi) + b2[...]
        if relu_out:
            z = jnp.maximum(z, 0.0)
        o[...] = z


def _mlp_call(relu_out):
    nblk = NPAD // BM
    blk = pl.BlockSpec((BM, 128), lambda i: (i, 0))
    full = lambda shape: pl.BlockSpec(shape, lambda i: (0, 0))
    return pl.pallas_call(
        functools.partial(_mlp_body, relu_out=relu_out),
        grid=(nblk,),
        in_specs=[blk] * 8 + [
            full((128, HIDP)), full((128, HIDP)), full((128, HIDP)),
            full((128, HIDP)), full((1, HIDP)),
            full((HIDP, 128)), full((HIDP, 128)), full((HIDP, 128)),
            full((1, 128)), full((1, 128)), full((1, 128)),
        ],
        out_specs=[blk, blk, blk],
        out_shape=[jax.ShapeDtypeStruct((NPAD, 128), jnp.float32)
                   for _ in range(3)],
    )


def _head_body(h0, h1, h2, gid2, hw_ref, hbias, out, acc):
    i = pl.program_id(0)
    g = gid2[...]
    oh = (g == lax.broadcasted_iota(jnp.int32, (1, NG), 1)).astype(jnp.float32)
    x = jnp.concatenate(
        [h0[...], h1[...], h2[...], jnp.ones((BM, 128), jnp.float32)], axis=1)
    p = lax.dot_general(oh, x, (((0,), (0,)), ((), ())),
                        preferred_element_type=jnp.float32,
                        precision=lax.Precision.HIGHEST)

    @pl.when(i == 0)
    def _():
        acc[...] = p

    @pl.when(i > 0)
    def _():
        acc[...] = acc[...] + p

    @pl.when(i == pl.num_programs(0) - 1)
    def _():
        a = acc[...]
        cnt = a[:, EMBP:EMBP + 1]
        pooled = a[:, :EMBP] / jnp.maximum(cnt, 1.0)
        out[...] = jnp.dot(pooled, hw_ref[...],
                           preferred_element_type=jnp.float32,
                           precision=lax.Precision.HIGHEST) + hbias[...]


_head_kernel = pl.pallas_call(
    _head_body,
    grid=(NPAD // BM,),
    in_specs=[
        pl.BlockSpec((BM, 128), lambda i: (i, 0)),
        pl.BlockSpec((BM, 128), lambda i: (i, 0)),
        pl.BlockSpec((BM, 128), lambda i: (i, 0)),
        pl.BlockSpec((BM, 1), lambda i: (i, 0)),
        pl.BlockSpec((EMBP, OUT_DIM), lambda i: (0, 0)),
        pl.BlockSpec((1, OUT_DIM), lambda i: (0, 0)),
    ],
    out_specs=pl.BlockSpec((NG, OUT_DIM), lambda i: (0, 0)),
    out_shape=jax.ShapeDtypeStruct((NG, OUT_DIM), jnp.float32),
    scratch_shapes=[pltpu.VMEM((NG, EMBP + 128), jnp.float32)],
)


def kernel(params, atom_type, chirality, edge_index, bond_type,
           bond_direction, graph_ids):
    f32 = jnp.float32

    atom_emb = params['atom_emb'].astype(f32)
    chi_emb = params['chi_emb'].astype(f32)
    ctab = (atom_emb[:, None, :] + chi_emb[None, :, :]).reshape(360, EMB)
    ctab = jnp.pad(ctab, ((0, 8), (0, EMBP - EMB)))
    ctabs = [ctab[:, k * 128:(k + 1) * 128] for k in range(NCH)]
    oh128 = jnp.pad(jnp.eye(32, dtype=f32), ((0, 0), (0, 96)))

    w1_l, ew1_l, b1_l, w2_l, b2_l = [], [], [], [], []
    for p in params['layers']:
        w1 = jnp.pad(p['W1'].astype(f32),
                     ((0, EMBP - EMB), (0, HIDP - HID)))
        w1_l.append([w1[k * 128:(k + 1) * 128] for k in range(NCH)])
        ec = (p['e_bond'][:, None, :] + p['e_dir'][None, :, :]).reshape(18, EMB)
        ew1 = ec.astype(f32) @ p['W1'].astype(f32)
        ew1_l.append(jnp.pad(ew1, ((0, 128 - 18), (0, HIDP - HID))))
        b1_l.append(jnp.pad(p['b1'].astype(f32), (0, HIDP - HID))[None, :])
        scale = p['bn_g'].astype(f32) / jnp.sqrt(1.0 + BN_EPS)
        w2f = p['W2'].astype(f32) * scale[None, :]
        b2f = p['b2'].astype(f32) * scale + p['bn_b'].astype(f32)
        w2f = jnp.pad(w2f, ((0, HIDP - HID), (0, EMBP - EMB)))
        b2f = jnp.pad(b2f, (0, EMBP - EMB))
        w2_l.append([w2f[:, k * 128:(k + 1) * 128] for k in range(NCH)])
        b2_l.append([b2f[None, k * 128:(k + 1) * 128] for k in range(NCH)])

    hw_p = jnp.pad(params['head_W'].astype(f32), ((0, EMBP - EMB), (0, 0)))
    hb_p = params['head_b'].astype(f32)[None, :]

    i32 = jnp.int32
    cidx = atom_type.astype(i32) * 3 + chirality.astype(i32)
    cidx3 = jnp.pad(cidx, (0, NPAD - N)).reshape(NTILES, NROWCH, 128)
    esh = (2, NTILES, ECHUNK, 128)
    src4 = jnp.pad(edge_index[0].astype(i32), (0, EPAD - E),
                   constant_values=NPAD - 1).reshape(esh)
    dst4 = jnp.pad(edge_index[1].astype(i32), (0, EPAD - E),
                   constant_values=NPAD - 1).reshape(esh)
    combo = bond_type.astype(i32) * 3 + bond_direction.astype(i32)
    combo4 = jnp.pad(combo, (0, EPAD - E)).reshape(esh)
    gid2 = jnp.pad(graph_ids.astype(i32), (0, NPAD - N),
                   constant_values=NG).reshape(NPAD, 1)

    h0, h1, h2, hista, histb = _init_kernel(*ctabs, cidx3, combo4, dst4,
                                            oh128)

    for l in range(5):
        parts = _spmm_kernel(h0, h1, h2, src4, dst4)
        h0, h1, h2 = _mlp_call(relu_out=(l != 4))(
            *parts, hista, histb, *w1_l[l], ew1_l[l], b1_l[l],
            *w2_l[l], *b2_l[l])

    return _head_kernel(h0, h1, h2, gid2, hw_p, hb_p)
